# baseline (speedup 1.0000x reference)
import functools

import jax
import jax.numpy as jnp
import ml_dtypes
import numpy as np
from jax.experimental import pallas as pl
from jax.experimental.pallas import tpu as pltpu


@functools.lru_cache(maxsize=None)
def _tap_incidence(k, pad, hin, win):
    hout = hin + 2 * pad - k + 1
    wout = win + 2 * pad - k + 1
    h2n, w2n = hout // 2, wout // 2
    q_arr = np.zeros((hin * win, 4, h2n * w2n, k * k), np.float32)

    h = np.arange(2 * h2n)[:, None, None, None]
    w = np.arange(2 * w2n)[None, :, None, None]
    kh = np.arange(k)[None, None, :, None]
    kw = np.arange(k)[None, None, None, :]
    hi = h + kh - pad
    wi = w + kw - pad
    valid = (hi >= 0) & (hi < hin) & (wi >= 0) & (wi < win)
    t = np.broadcast_to(kh * k + kw, valid.shape)
    p = (hi * win + wi)
    q = (h % 2) * 2 + (w % 2)
    s = (h // 2) * w2n + (w // 2)
    q_b, s_b = np.broadcast_to(q, valid.shape), np.broadcast_to(s, valid.shape)
    q_arr[p[valid], q_b[valid], s_b[valid], t[valid]] = 1.0
    return q_arr.reshape(hin * win * 4 * h2n * w2n, k * k).astype(
        ml_dtypes.bfloat16)


def _densify(wconv, k, pad, hin, win):
    cin, cout = wconv.shape[2], wconv.shape[3]
    q_flat = jnp.asarray(_tap_incidence(k, pad, hin, win))
    rows, tt = q_flat.shape
    npos = hin * win
    wr = wconv.astype(jnp.bfloat16).reshape(tt, cin, cout)
    pieces = [
        jnp.dot(q_flat, wr[:, i, :],
                preferred_element_type=jnp.bfloat16).reshape(npos, -1)
        for i in range(cin)
    ]
    if cin == 1:
        return pieces[0]
    return jnp.stack(pieces, axis=1).reshape(npos * cin, -1)


def _fused_body(x_ref, w1_ref, b1_ref, w2_ref, b2_ref, w3_ref, b3_ref,
                wf1_ref, bf1_ref, wf2_ref, bf2_ref, out_ref):
    def conv_pool(x, w_ref, b_row):
        nq = w_ref.shape[1] // 4
        m = None
        for q in range(4):
            a = jnp.dot(x, w_ref[:, q * nq:(q + 1) * nq],
                        preferred_element_type=jnp.float32)
            m = a if m is None else jnp.maximum(m, a)
        return jnp.maximum(m + b_row, 0.0).astype(jnp.bfloat16)

    p1 = conv_pool(x_ref[...], w1_ref, b1_ref[...])
    p2 = conv_pool(p1, w2_ref, b2_ref[...])
    p3 = conv_pool(p2, w3_ref, b3_ref[...])
    h1 = jnp.maximum(
        jnp.dot(p3, wf1_ref[...], preferred_element_type=jnp.float32)
        + bf1_ref[...], 0.0).astype(jnp.bfloat16)
    out_ref[...] = (jnp.dot(h1, wf2_ref[...],
                            preferred_element_type=jnp.float32)
                    + bf2_ref[...])


def kernel(w1, b1, w2, b2, w3, b3, wfc1, bfc1, wfc2, bfc2, x_nchw):
    n, cin, h, w = x_nchw.shape
    c3 = w3.shape[3]
    classes = wfc2.shape[1]

    h1o, w1o = (h + 8 - 4) // 2, (w + 8 - 4) // 2
    h2o, w2o = (h1o + 4 - 4) // 2, (w1o + 4 - 4) // 2
    h3o, w3o = (h2o + 2 - 2) // 2, (w2o + 2 - 2) // 2
    pcnt = h3o * w3o
    feat = c3 * pcnt

    w1d = _densify(w1, 5, 4, h, w)
    w2d = _densify(w2, 5, 2, h1o, w1o)
    w3d = _densify(w3, 3, 1, h2o, w2o)
    b1t = jnp.tile(b1.astype(jnp.float32), h1o * w1o).reshape(1, -1)
    b2t = jnp.tile(b2.astype(jnp.float32), h2o * w2o).reshape(1, -1)
    b3t = jnp.tile(b3.astype(jnp.float32), h3o * w3o).reshape(1, -1)

    wf1 = (wfc1.astype(jnp.float32).reshape(c3, pcnt, feat)
           .transpose(1, 0, 2).reshape(feat, feat).astype(jnp.bfloat16))
    bf1 = bfc1.astype(jnp.float32).reshape(1, feat)
    ncls = 128
    wf2 = jnp.pad(wfc2.astype(jnp.float32), ((0, 0), (0, ncls - classes))
                  ).astype(jnp.bfloat16)
    bf2 = jnp.pad(bfc2.astype(jnp.float32), (0, ncls - classes)).reshape(1, ncls)

    x_flat = (jnp.transpose(x_nchw, (0, 2, 3, 1))
              .reshape(n, h * w * cin).astype(jnp.bfloat16))

    bsz = 256
    while n % bsz:
        bsz //= 2
    grid = (n // bsz,)

    def whole(arr):
        return pl.BlockSpec(arr.shape, lambda i, _nd=arr.ndim: (0,) * _nd,
                            memory_space=pltpu.MemorySpace.VMEM)

    flops = 2 * n * (w1d.shape[0] * w1d.shape[1] + w2d.shape[0] * w2d.shape[1]
                     + w3d.shape[0] * w3d.shape[1] + feat * feat + feat * ncls)
    bytes_accessed = (x_flat.size * 2 + w1d.size * 2 + w2d.size * 2
                      + w3d.size * 2 + wf1.size * 2 + wf2.size * 2
                      + n * ncls * 4)

    out = pl.pallas_call(
        _fused_body,
        out_shape=jax.ShapeDtypeStruct((n, ncls), jnp.float32),
        grid=grid,
        in_specs=[
            pl.BlockSpec((bsz, h * w * cin), lambda i: (i, 0),
                         memory_space=pltpu.MemorySpace.VMEM),
            whole(w1d), whole(b1t), whole(w2d), whole(b2t),
            whole(w3d), whole(b3t), whole(wf1), whole(bf1),
            whole(wf2), whole(bf2),
        ],
        out_specs=pl.BlockSpec((bsz, ncls), lambda i: (i, 0),
                               memory_space=pltpu.MemorySpace.VMEM),
        compiler_params=pltpu.CompilerParams(
            dimension_semantics=("parallel",)),
        cost_estimate=pl.CostEstimate(flops=flops, transcendentals=0,
                                      bytes_accessed=bytes_accessed),
    )(x_flat, w1d, b1t, w2d, b2t, w3d, b3t, wf1, bf1, wf2, bf2)
    return out[:, :classes]


# speedup vs baseline: 2.1901x; 2.1901x over previous
import functools

import jax
import jax.numpy as jnp
import numpy as np
from jax.experimental import pallas as pl
from jax.experimental.pallas import tpu as pltpu


@functools.lru_cache(maxsize=None)
def _tap_incidence(k, pad, hin, win):
    hout = hin + 2 * pad - k + 1
    wout = win + 2 * pad - k + 1
    h2n, w2n = hout // 2, wout // 2
    q_arr = np.zeros((k * k, hin * win, 4, h2n * w2n), np.float32)

    h = np.arange(2 * h2n)[:, None, None, None]
    w = np.arange(2 * w2n)[None, :, None, None]
    kh = np.arange(k)[None, None, :, None]
    kw = np.arange(k)[None, None, None, :]
    hi = h + kh - pad
    wi = w + kw - pad
    valid = (hi >= 0) & (hi < hin) & (wi >= 0) & (wi < win)
    t = np.broadcast_to(kh * k + kw, valid.shape)
    p = (hi * win + wi)
    q = (h % 2) * 2 + (w % 2)
    s = (h // 2) * w2n + (w // 2)
    q_b, s_b = np.broadcast_to(q, valid.shape), np.broadcast_to(s, valid.shape)
    q_arr[t[valid], p[valid], q_b[valid], s_b[valid]] = 1.0
    return q_arr


def _densify(wconv, k, pad, hin, win):
    cin, cout = wconv.shape[2], wconv.shape[3]
    q_arr = jnp.asarray(_tap_incidence(k, pad, hin, win), jnp.bfloat16)
    wr = wconv.astype(jnp.bfloat16).reshape(k * k, cin, cout)
    dense = jnp.einsum("tpqs,tio->ipqos", q_arr, wr,
                       preferred_element_type=jnp.bfloat16)
    npos = q_arr.shape[1]
    npool = q_arr.shape[3]
    return dense.reshape(cin * npos, 4 * cout * npool)


def _fused_body(x_ref, w1_ref, b1_ref, w2_ref, b2_ref, w3_ref, b3_ref,
                wf1_ref, bf1_ref, wf2_ref, bf2_ref, out_ref):
    def conv_pool(x, w_ref, b_row):
        nq = w_ref.shape[1] // 4
        m = None
        for q in range(4):
            a = jnp.dot(x, w_ref[:, q * nq:(q + 1) * nq],
                        preferred_element_type=jnp.float32)
            m = a if m is None else jnp.maximum(m, a)
        return jnp.maximum(m + b_row, 0.0).astype(jnp.bfloat16)

    p1 = conv_pool(x_ref[...], w1_ref, b1_ref[...])
    p2 = conv_pool(p1, w2_ref, b2_ref[...])
    p3 = conv_pool(p2, w3_ref, b3_ref[...])
    h1 = jnp.maximum(
        jnp.dot(p3, wf1_ref[...], preferred_element_type=jnp.float32)
        + bf1_ref[...], 0.0).astype(jnp.bfloat16)
    out_ref[...] = (jnp.dot(h1, wf2_ref[...],
                            preferred_element_type=jnp.float32)
                    + bf2_ref[...])


def kernel(w1, b1, w2, b2, w3, b3, wfc1, bfc1, wfc2, bfc2, x_nchw):
    n, cin, h, w = x_nchw.shape
    classes = wfc2.shape[1]

    h1o, w1o = (h + 8 - 4) // 2, (w + 8 - 4) // 2
    h2o, w2o = (h1o + 4 - 4) // 2, (w1o + 4 - 4) // 2
    h3o, w3o = (h2o + 2 - 2) // 2, (w2o + 2 - 2) // 2
    feat = w3.shape[3] * h3o * w3o

    w1d = _densify(w1, 5, 4, h, w)
    w2d = _densify(w2, 5, 2, h1o, w1o)
    w3d = _densify(w3, 3, 1, h2o, w2o)
    b1t = jnp.repeat(b1.astype(jnp.float32), h1o * w1o).reshape(1, -1)
    b2t = jnp.repeat(b2.astype(jnp.float32), h2o * w2o).reshape(1, -1)
    b3t = jnp.repeat(b3.astype(jnp.float32), h3o * w3o).reshape(1, -1)

    wf1 = wfc1.astype(jnp.bfloat16)
    bf1 = bfc1.astype(jnp.float32).reshape(1, feat)
    ncls = 128
    wf2 = jnp.pad(wfc2.astype(jnp.float32), ((0, 0), (0, ncls - classes))
                  ).astype(jnp.bfloat16)
    bf2 = jnp.pad(bfc2.astype(jnp.float32), (0, ncls - classes)).reshape(1, ncls)

    x_flat = (jnp.transpose(x_nchw, (0, 2, 3, 1))
              .reshape(n, h * w * cin).astype(jnp.bfloat16))

    bsz = 512
    while n % bsz:
        bsz //= 2
    grid = (n // bsz,)

    def whole(arr):
        return pl.BlockSpec(arr.shape, lambda i, _nd=arr.ndim: (0,) * _nd,
                            memory_space=pltpu.MemorySpace.VMEM)

    flops = 2 * n * (w1d.shape[0] * w1d.shape[1] + w2d.shape[0] * w2d.shape[1]
                     + w3d.shape[0] * w3d.shape[1] + feat * feat + feat * ncls)
    bytes_accessed = (x_flat.size * 2 + w1d.size * 2 + w2d.size * 2
                      + w3d.size * 2 + wf1.size * 2 + wf2.size * 2
                      + n * ncls * 4)

    out = pl.pallas_call(
        _fused_body,
        out_shape=jax.ShapeDtypeStruct((n, ncls), jnp.float32),
        grid=grid,
        in_specs=[
            pl.BlockSpec((bsz, h * w * cin), lambda i: (i, 0),
                         memory_space=pltpu.MemorySpace.VMEM),
            whole(w1d), whole(b1t), whole(w2d), whole(b2t),
            whole(w3d), whole(b3t), whole(wf1), whole(bf1),
            whole(wf2), whole(bf2),
        ],
        out_specs=pl.BlockSpec((bsz, ncls), lambda i: (i, 0),
                               memory_space=pltpu.MemorySpace.VMEM),
        compiler_params=pltpu.CompilerParams(
            dimension_semantics=("parallel",)),
        cost_estimate=pl.CostEstimate(flops=flops, transcendentals=0,
                                      bytes_accessed=bytes_accessed),
    )(x_flat, w1d, b1t, w2d, b2t, w3d, b3t, wf1, bf1, wf2, bf2)
    return out[:, :classes]
